# baseline (speedup 1.0000x reference)
import functools

import jax
import jax.numpy as jnp
from jax import lax
from jax.experimental import pallas as pl
from jax.experimental.pallas import tpu as pltpu
from jax.experimental.pallas import tpu_sc as plsc

N_NODES_ = 100000
N_TYPES_ = 64
D_ = 128
BLK_ = 10000

NW_ = 32
CHUNK_ = 400
KMAX_ = 8
N_PAD_ = NW_ * CHUNK_ * KMAX_
NCHUNK_ = N_NODES_ // CHUNK_


def _tc_onehot_body(ide_ref, ido_ref, eye_ref, oh_ref):
    ide = ide_ref[0]
    ido = ido_ref[0]
    iota = lax.broadcasted_iota(jnp.int32, (N_TYPES_, BLK_), 0)
    oh2_t = jnp.concatenate(
        [(ide == iota), (ido == iota)], axis=0).astype(jnp.float32)
    oh_ref[...] = lax.dot_general(
        oh2_t, eye_ref[...], (((0,), (0,)), ((), ())),
        preferred_element_type=jnp.float32)


def _sc_gather_body(w_hbm, idx_hbm, out_hbm, w_sh, idx_vs, rows_vs, gsem,
                    ssems):
    wid = lax.axis_index("s") * 2 + lax.axis_index("c")

    def chunk_id(k):
        return wid + NW_ * k

    def guarded(k, go):
        if k == KMAX_ - 1:
            pl.when(chunk_id(k) < NCHUNK_)(go)
        else:
            go()

    def store(k, fire):
        def go():
            c = pltpu.make_async_copy(
                rows_vs[k % 2],
                out_hbm.at[pl.ds(chunk_id(k) * CHUNK_, CHUNK_)],
                ssems[k % 2])
            c.start() if fire else c.wait()

        guarded(k, go)

    @pl.when(lax.axis_index("s") == 0)
    def _():
        pltpu.sync_copy(w_hbm, w_sh)

    for k in range(KMAX_):
        guarded(k, lambda k=k: pltpu.sync_copy(
            idx_hbm.at[pl.ds(chunk_id(k) * CHUNK_, CHUNK_)], idx_vs[k]))
    plsc.subcore_barrier()

    for k in range(KMAX_):
        if k >= 2:
            store(k - 2, fire=False)
        guarded(k, lambda k=k: pltpu.async_copy(
            w_sh.at[idx_vs[k]], rows_vs[k % 2], gsem).wait())
        store(k, fire=True)

    store(KMAX_ - 2, fire=False)
    store(KMAX_ - 1, fire=False)


@functools.partial(
    pl.kernel,
    mesh=plsc.VectorSubcoreMesh(core_axis_name="c", subcore_axis_name="s"),
    cost_estimate=pl.CostEstimate(
        flops=0, transcendentals=0, bytes_accessed=103_000_000),
    out_type=jax.ShapeDtypeStruct((N_NODES_, D_), jnp.float32),
    scratch_types=(
        [pltpu.VMEM_SHARED((N_TYPES_, D_), jnp.float32)]
        + [pltpu.VMEM((CHUNK_,), jnp.int32) for _ in range(KMAX_)]
        + [
            pltpu.VMEM((CHUNK_, D_), jnp.float32),
            pltpu.VMEM((CHUNK_, D_), jnp.float32),
            pltpu.SemaphoreType.DMA,
            pltpu.SemaphoreType.DMA,
            pltpu.SemaphoreType.DMA,
        ]
    ),
)
def _sc_gather(w_hbm, idx_hbm, out_hbm, *scratch):
    w_sh = scratch[0]
    idx_vs = list(scratch[1:1 + KMAX_])
    r0, r1, gsem, s0, s1 = scratch[1 + KMAX_:]
    _sc_gather_body(w_hbm, idx_hbm, out_hbm, w_sh, idx_vs, [r0, r1], gsem,
                    [s0, s1])


def kernel(node_atom, W, b):
    idx = node_atom.astype(jnp.int32)
    table = W + b[None, :]

    half = N_NODES_ // 2
    grid = half // BLK_
    ide3 = idx[0::2].reshape(grid, 1, BLK_)
    ido3 = idx[1::2].reshape(grid, 1, BLK_)
    eye2 = jnp.eye(2 * N_TYPES_, dtype=jnp.float32)
    packed = pl.pallas_call(
        _tc_onehot_body,
        grid=(grid,),
        in_specs=[
            pl.BlockSpec((1, 1, BLK_), lambda i: (i, 0, 0)),
            pl.BlockSpec((1, 1, BLK_), lambda i: (i, 0, 0)),
            pl.BlockSpec((2 * N_TYPES_, 2 * N_TYPES_), lambda i: (0, 0)),
        ],
        out_specs=pl.BlockSpec((BLK_, 2 * N_TYPES_), lambda i: (i, 0)),
        out_shape=jax.ShapeDtypeStruct((half, 2 * N_TYPES_), jnp.float32),
    )(ide3, ido3, eye2)
    oh = packed.reshape(N_NODES_, N_TYPES_)
    emb = _sc_gather(table, idx)
    return (emb, oh, oh)


# speedup vs baseline: 1.5395x; 1.5395x over previous
import functools

import jax
import jax.numpy as jnp
from jax import lax
from jax.experimental import pallas as pl
from jax.experimental.pallas import tpu as pltpu
from jax.experimental.pallas import tpu_sc as plsc

N_NODES_ = 100000
N_TYPES_ = 64
D_ = 128
BLK_ = 10000

NW_ = 32
CHUNK_ = 400
KMAX_ = 8
N_PAD_ = NW_ * CHUNK_ * KMAX_
NCHUNK_ = N_NODES_ // CHUNK_


def _tc_onehot_body(idx_ref, eye_ref, oh_ref):
    idx = idx_ref[0]
    iota = lax.broadcasted_iota(jnp.int32, (N_TYPES_, BLK_), 0)
    onehot_t = (idx == iota).astype(jnp.float32)
    oh_ref[...] = lax.dot_general(
        onehot_t, eye_ref[...], (((0,), (0,)), ((), ())),
        preferred_element_type=jnp.float32)


def _sc_gather_body(w_hbm, idx_hbm, out_hbm, w_sh, idx_vs, rows_vs, gsem,
                    ssems):
    wid = lax.axis_index("s") * 2 + lax.axis_index("c")

    def chunk_id(k):
        return wid + NW_ * k

    def guarded(k, go):
        if k == KMAX_ - 1:
            pl.when(chunk_id(k) < NCHUNK_)(go)
        else:
            go()

    def store(k, fire):
        def go():
            c = pltpu.make_async_copy(
                rows_vs[k % 2],
                out_hbm.at[pl.ds(chunk_id(k) * CHUNK_, CHUNK_)],
                ssems[k % 2])
            c.start() if fire else c.wait()

        guarded(k, go)

    @pl.when(lax.axis_index("s") == 0)
    def _():
        pltpu.sync_copy(w_hbm, w_sh)

    for k in range(KMAX_):
        guarded(k, lambda k=k: pltpu.sync_copy(
            idx_hbm.at[pl.ds(chunk_id(k) * CHUNK_, CHUNK_)], idx_vs[k]))
    plsc.subcore_barrier()

    for k in range(KMAX_):
        if k >= 2:
            store(k - 2, fire=False)
        guarded(k, lambda k=k: pltpu.async_copy(
            w_sh.at[idx_vs[k]], rows_vs[k % 2], gsem).wait())
        store(k, fire=True)

    store(KMAX_ - 2, fire=False)
    store(KMAX_ - 1, fire=False)


@functools.partial(
    pl.kernel,
    mesh=plsc.VectorSubcoreMesh(core_axis_name="c", subcore_axis_name="s"),
    cost_estimate=pl.CostEstimate(
        flops=0, transcendentals=0, bytes_accessed=103_000_000),
    out_type=jax.ShapeDtypeStruct((N_NODES_, D_), jnp.float32),
    scratch_types=(
        [pltpu.VMEM_SHARED((N_TYPES_, D_), jnp.float32)]
        + [pltpu.VMEM((CHUNK_,), jnp.int32) for _ in range(KMAX_)]
        + [
            pltpu.VMEM((CHUNK_, D_), jnp.float32),
            pltpu.VMEM((CHUNK_, D_), jnp.float32),
            pltpu.SemaphoreType.DMA,
            pltpu.SemaphoreType.DMA,
            pltpu.SemaphoreType.DMA,
        ]
    ),
)
def _sc_gather(w_hbm, idx_hbm, out_hbm, *scratch):
    w_sh = scratch[0]
    idx_vs = list(scratch[1:1 + KMAX_])
    r0, r1, gsem, s0, s1 = scratch[1 + KMAX_:]
    _sc_gather_body(w_hbm, idx_hbm, out_hbm, w_sh, idx_vs, [r0, r1], gsem,
                    [s0, s1])


def kernel(node_atom, W, b):
    idx = node_atom.astype(jnp.int32)
    table = W + b[None, :]

    idx3 = idx.reshape(N_NODES_ // BLK_, 1, BLK_)
    eye = jnp.eye(N_TYPES_, dtype=jnp.float32)
    oh = pl.pallas_call(
        _tc_onehot_body,
        grid=(N_NODES_ // BLK_,),
        in_specs=[
            pl.BlockSpec((1, 1, BLK_), lambda i: (i, 0, 0)),
            pl.BlockSpec((N_TYPES_, N_TYPES_), lambda i: (0, 0)),
        ],
        out_specs=pl.BlockSpec((BLK_, N_TYPES_), lambda i: (i, 0)),
        out_shape=jax.ShapeDtypeStruct((N_NODES_, N_TYPES_), jnp.float32),
    )(idx3, eye)
    emb = _sc_gather(table, idx)
    return (emb, oh, oh)
